# baseline (speedup 1.0000x reference)
import jax
import jax.numpy as jnp
from jax import lax
from jax.experimental import pallas as pl
from jax.experimental.pallas import tpu as pltpu
from jax.experimental.pallas import tpu_sc as plsc

_B, _C, _W, _H = 4, 96, 224, 224
_G = 8
_CH = 24
_GPU = _CH // _G
_WT = 8
_NWT = _W // _WT
_UNITS = _B * (_C // _CH) * _NWT
_NWORKERS = 32
_UPW = _UNITS // _NWORKERS
_LANES = 16
_JH = _H // _LANES


def _body(x_hbm, out_hbm, buf0, buf1, si0, si1, so0, so1):
    cid = lax.axis_index("c")
    sid = lax.axis_index("s")
    wid = sid * 2 + cid
    bufs, sins, souts = (buf0, buf1), (si0, si1), (so0, so1)

    def slices(u):
        uid = wid * _UPW + u
        b = uid // ((_C // _CH) * _NWT)
        rem = uid % ((_C // _CH) * _NWT)
        c0 = (rem // _NWT) * _CH
        w0 = (rem % _NWT) * _WT
        return b, c0, w0

    def start_in(u, p):
        b, c0, w0 = slices(u)
        return pltpu.async_copy(
            x_hbm.at[b, pl.ds(c0, _CH), pl.ds(w0, _WT), :], bufs[p], sins[p])

    def start_out(u, p):
        b, c0, w0 = slices(u)
        return pltpu.async_copy(
            bufs[p], out_hbm.at[b, pl.ds(c0, _CH), pl.ds(w0, _WT), :],
            souts[p])

    def compute(p):
        buf = bufs[p]

        @plsc.parallel_loop(0, _GPU * _WT * _JH, unroll=4)
        def _vec_step(t):
            g = t // (_WT * _JH)
            rem = t % (_WT * _JH)
            w = rem // _JH
            col = (rem % _JH) * _LANES
            v = [buf[g * _G + i, w, pl.ds(col, _LANES)] for i in range(_G)]
            m = v[0]
            for i in range(1, _G):
                m = jnp.maximum(m, v[i])
            r = jnp.maximum(m, 0.0)
            zero = jnp.zeros((_LANES,), jnp.float32)
            pmax = jnp.full((_LANES,), -jnp.inf, jnp.float32)
            for i in range(_G):
                sel = jnp.logical_and(v[i] == m, pmax < m)
                if i < _G - 1:
                    pmax = jnp.maximum(pmax, v[i])
                buf[g * _G + i, w, pl.ds(col, _LANES)] = (
                    jnp.where(sel, r, zero))

    in_h = [None] * _UPW
    out_h = [None] * _UPW
    in_h[0] = start_in(0, 0)
    for u in range(_UPW):
        p = u % 2
        in_h[u].wait()
        if u + 1 < _UPW:
            if u >= 1:
                out_h[u - 1].wait()
            in_h[u + 1] = start_in(u + 1, 1 - p)
        compute(p)
        out_h[u] = start_out(u, p)
    out_h[_UPW - 2].wait()
    out_h[_UPW - 1].wait()


def kernel(x):
    assert x.shape == (_B, _C, _W, _H) and x.dtype == jnp.float32
    mesh = plsc.VectorSubcoreMesh(core_axis_name="c", subcore_axis_name="s")
    return pl.kernel(
        _body,
        out_type=jax.ShapeDtypeStruct((_B, _C, _W, _H), jnp.float32),
        mesh=mesh,
        scratch_types=[
            pltpu.VMEM((_CH, _WT, _H), jnp.float32),
            pltpu.VMEM((_CH, _WT, _H), jnp.float32),
            pltpu.SemaphoreType.DMA,
            pltpu.SemaphoreType.DMA,
            pltpu.SemaphoreType.DMA,
            pltpu.SemaphoreType.DMA,
        ],
    )(x)


# speedup vs baseline: 2.3943x; 2.3943x over previous
import jax
import jax.numpy as jnp
from jax import lax
from jax.experimental import pallas as pl
from jax.experimental.pallas import tpu as pltpu
from jax.experimental.pallas import tpu_sc as plsc

_B, _C, _W, _H = 4, 96, 224, 224
_G = 8
_CH = 24
_GPU = _CH // _G
_WT = 8
_NWT = _W // _WT
_UNITS = _B * (_C // _CH) * _NWT
_NWORKERS = 32
_UPW = _UNITS // _NWORKERS
_LANES = 16
_JH = _H // _LANES


def _body(x_hbm, out_hbm, buf0, buf1, si0, si1, so0, so1):
    cid = lax.axis_index("c")
    sid = lax.axis_index("s")
    wid = sid * 2 + cid
    bufs, sins, souts = (buf0, buf1), (si0, si1), (so0, so1)

    def slices(u):
        uid = wid * _UPW + u
        b = uid // ((_C // _CH) * _NWT)
        rem = uid % ((_C // _CH) * _NWT)
        c0 = (rem // _NWT) * _CH
        w0 = (rem % _NWT) * _WT
        return b, c0, w0

    def start_in(u, p):
        b, c0, w0 = slices(u)
        return pltpu.async_copy(
            x_hbm.at[b, pl.ds(c0, _CH), pl.ds(w0, _WT), :], bufs[p], sins[p])

    def start_out(u, p):
        b, c0, w0 = slices(u)
        return pltpu.async_copy(
            bufs[p], out_hbm.at[b, pl.ds(c0, _CH), pl.ds(w0, _WT), :],
            souts[p])

    def compute(p):
        buf = bufs[p]

        @plsc.parallel_loop(0, _GPU * _WT * _JH, unroll=3)
        def _vec_step(t):
            g = t // (_WT * _JH)
            rem = t % (_WT * _JH)
            w = rem // _JH
            col = (rem % _JH) * _LANES
            v = [buf[g * _G + i, w, pl.ds(col, _LANES)] for i in range(_G)]
            p01 = jnp.maximum(v[0], v[1])
            p23 = jnp.maximum(v[2], v[3])
            p45 = jnp.maximum(v[4], v[5])
            p67 = jnp.maximum(v[6], v[7])
            q03 = jnp.maximum(p01, p23)
            q47 = jnp.maximum(p45, p67)
            m = jnp.maximum(q03, q47)
            r = jnp.maximum(m, 0.0)
            zero = jnp.zeros((_LANES,), jnp.float32)
            pf = [None, v[0], p01, jnp.maximum(p01, v[2]), q03,
                  jnp.maximum(q03, v[4]), jnp.maximum(q03, p45), None]
            pf[7] = jnp.maximum(pf[6], v[6])
            for i in range(_G):
                eq = v[i] == m
                sel = eq if i == 0 else jnp.logical_and(eq, pf[i] < m)
                buf[g * _G + i, w, pl.ds(col, _LANES)] = (
                    jnp.where(sel, r, zero))

    in_h = [None] * _UPW
    out_h = [None] * _UPW
    in_h[0] = start_in(0, 0)
    for u in range(_UPW):
        p = u % 2
        in_h[u].wait()
        if u + 1 < _UPW:
            if u >= 1:
                out_h[u - 1].wait()
            in_h[u + 1] = start_in(u + 1, 1 - p)
        compute(p)
        out_h[u] = start_out(u, p)
    out_h[_UPW - 2].wait()
    out_h[_UPW - 1].wait()


def kernel(x):
    assert x.shape == (_B, _C, _W, _H) and x.dtype == jnp.float32
    mesh = plsc.VectorSubcoreMesh(core_axis_name="c", subcore_axis_name="s")
    return pl.kernel(
        _body,
        out_type=jax.ShapeDtypeStruct((_B, _C, _W, _H), jnp.float32),
        mesh=mesh,
        scratch_types=[
            pltpu.VMEM((_CH, _WT, _H), jnp.float32),
            pltpu.VMEM((_CH, _WT, _H), jnp.float32),
            pltpu.SemaphoreType.DMA,
            pltpu.SemaphoreType.DMA,
            pltpu.SemaphoreType.DMA,
            pltpu.SemaphoreType.DMA,
        ],
    )(x)
